# baseline (speedup 1.0000x reference)
import functools

import jax
import jax.numpy as jnp
from jax import lax
from jax.experimental import pallas as pl
from jax.experimental.pallas import tpu as pltpu
from jax.experimental.pallas import tpu_sc as plsc

_B = 4096
_E = 8
_G = _E * _E
_BLK = 32
_P = _B + _G * _BLK
_NBLK = _P // _BLK


@functools.cache
def _make_sc_gather(V, D, N):
    info = plsc.get_sparse_core_info()
    NC, NS = info.num_cores, info.num_subcores
    NW = NC * NS
    assert N % NW == 0
    rows_pw = N // NW
    cap = min(128, rows_pw, (192 * 1024) // (D * 4))
    chunk = 8
    for c in range(8, cap + 1, 8):
        if rows_pw % c == 0:
            chunk = c
    nch = rows_pw // chunk
    nbuf = 2 if nch > 1 else 1
    mesh = plsc.VectorSubcoreMesh(core_axis_name="c", subcore_axis_name="s")

    @functools.partial(
        pl.kernel,
        out_type=jax.ShapeDtypeStruct((N, D), jnp.float32),
        mesh=mesh,
        scratch_types=[
            pltpu.VMEM((rows_pw,), jnp.int32),
            pltpu.VMEM((nbuf, chunk, D), jnp.float32),
        ] + [pltpu.SemaphoreType.DMA] * nbuf,
    )
    def gather_kernel(table_hbm, idx_hbm, out_hbm, idx_v, rows_v, *sems):
        wid = lax.axis_index("s") * NC + lax.axis_index("c")
        base = wid * rows_pw
        pltpu.sync_copy(idx_hbm.at[pl.ds(base, rows_pw)], idx_v)
        inflight = [None] * nbuf
        for c in range(nch):
            b = c % nbuf
            if inflight[c % nbuf] is not None:
                inflight[b].wait()
                prev = c - nbuf
                pltpu.sync_copy(
                    rows_v.at[b], out_hbm.at[pl.ds(base + prev * chunk, chunk)])
            inflight[b] = pltpu.async_copy(
                table_hbm.at[idx_v.at[pl.ds(c * chunk, chunk)]],
                rows_v.at[b], sems[b])
        for c in range(max(0, nch - nbuf), nch):
            b = c % nbuf
            inflight[b].wait()
            pltpu.sync_copy(
                rows_v.at[b], out_hbm.at[pl.ds(base + c * chunk, chunk)])

    return gather_kernel


def _mlp0_body(be_ref, act_ref, x_ref, w1_ref, b1_ref, w2_ref, b2_ref, o_ref):
    i = pl.program_id(0)

    @pl.when(act_ref[i] != 0)
    def _():
        e = be_ref[i]
        h = jnp.dot(x_ref[...], w1_ref[e], preferred_element_type=jnp.float32)
        h = jnp.maximum(h + b1_ref[e], 0.0)
        y = jnp.dot(h, w2_ref[e], preferred_element_type=jnp.float32)
        o_ref[...] = jnp.maximum(y + b2_ref[e], 0.0)


def _mlp0_call(be, act, f01_s, w1, b1, w2, b2):
    grid_spec = pltpu.PrefetchScalarGridSpec(
        num_scalar_prefetch=2,
        grid=(_NBLK,),
        in_specs=[
            pl.BlockSpec((_BLK, 128), lambda i, be, act: (i, 0)),
            pl.BlockSpec((_E, 128, 1024), lambda i, be, act: (0, 0, 0)),
            pl.BlockSpec((_E, 1, 1024), lambda i, be, act: (0, 0, 0)),
            pl.BlockSpec((_E, 1024, 1024), lambda i, be, act: (0, 0, 0)),
            pl.BlockSpec((_E, 1, 1024), lambda i, be, act: (0, 0, 0)),
        ],
        out_specs=pl.BlockSpec((_BLK, 1024), lambda i, be, act: (i, 0)),
    )
    return pl.pallas_call(
        _mlp0_body,
        grid_spec=grid_spec,
        out_shape=jax.ShapeDtypeStruct((_P, 1024), jnp.float32),
    )(be, act, f01_s, w1, b1.reshape(_E, 1, 1024), w2, b2.reshape(_E, 1, 1024))


def _mlp1_body(be_ref, act_ref, x0_ref, f1_ref, pw_ref, pb_ref, w1a_ref,
               w1b_ref, b1_ref, w2_ref, b2_ref, o_ref):
    i = pl.program_id(0)

    @pl.when(act_ref[i] != 0)
    def _():
        pre = jnp.dot(f1_ref[...], pw_ref[0], preferred_element_type=jnp.float32)
        pre = jnp.maximum(pre + pb_ref[0], 0.0)
        h = jnp.dot(x0_ref[...], w1a_ref[0], preferred_element_type=jnp.float32)
        h = h + jnp.dot(pre, w1b_ref[0], preferred_element_type=jnp.float32)
        h = jnp.maximum(h + b1_ref[0], 0.0)
        y = jnp.dot(h, w2_ref[0], preferred_element_type=jnp.float32)
        o_ref[...] = y + b2_ref[0]


def _mlp1_call(be, act, x0_s, f01_s, pw, pb, w1, b1, w2, b2):
    w1a = w1[:, :1024, :]
    w1b = w1[:, 1024:, :]
    grid_spec = pltpu.PrefetchScalarGridSpec(
        num_scalar_prefetch=2,
        grid=(_NBLK,),
        in_specs=[
            pl.BlockSpec((_BLK, 1024), lambda i, be, act: (i, 0)),
            pl.BlockSpec((_BLK, 128), lambda i, be, act: (i, 1)),
            pl.BlockSpec((1, 128, 512), lambda i, be, act: (be[i], 0, 0)),
            pl.BlockSpec((1, 1, 512), lambda i, be, act: (be[i], 0, 0)),
            pl.BlockSpec((1, 1024, 1024), lambda i, be, act: (be[i], 0, 0)),
            pl.BlockSpec((1, 512, 1024), lambda i, be, act: (be[i], 0, 0)),
            pl.BlockSpec((1, 1, 1024), lambda i, be, act: (be[i], 0, 0)),
            pl.BlockSpec((1, 1024, 512), lambda i, be, act: (be[i], 0, 0)),
            pl.BlockSpec((1, 1, 512), lambda i, be, act: (be[i], 0, 0)),
        ],
        out_specs=pl.BlockSpec((_BLK, 512), lambda i, be, act: (i, 0)),
    )
    return pl.pallas_call(
        _mlp1_body,
        grid_spec=grid_spec,
        out_shape=jax.ShapeDtypeStruct((_P, 512), jnp.float32),
    )(be, act, x0_s, f01_s, pw, pb.reshape(_E, 1, 512), w1a, w1b,
      b1.reshape(_E, 1, 1024), w2, b2.reshape(_E, 1, 512))


def _routing(oh0, oh1):
    koh = (oh1[:, :, None] * oh0[:, None, :]).reshape(_B, _G)
    counts = jnp.sum(koh, axis=0).astype(jnp.int32)
    rank_all = jnp.cumsum(koh, axis=0) - koh
    key = jnp.argmax(koh, axis=1).astype(jnp.int32)
    rank = jnp.take_along_axis(rank_all, key[:, None], axis=1)[:, 0]
    rank = rank.astype(jnp.int32)
    pc = ((counts + _BLK - 1) // _BLK) * _BLK
    starts = jnp.concatenate(
        [jnp.zeros((1,), jnp.int32), jnp.cumsum(pc)[:-1].astype(jnp.int32)])
    dest = starts[key] + rank
    row_ids = jnp.zeros((_P,), jnp.int32).at[dest].set(
        jnp.arange(_B, dtype=jnp.int32))
    r = jnp.arange(_NBLK, dtype=jnp.int32) * _BLK
    total = jnp.sum(pc)
    g = jnp.clip(
        jnp.searchsorted(starts, r, side="right").astype(jnp.int32) - 1,
        0, _G - 1)
    act = (r < total).astype(jnp.int32)
    last_g = jnp.maximum(
        jnp.max(jnp.where(counts > 0, jnp.arange(_G, dtype=jnp.int32), -1)), 0)
    g = jnp.where(act == 1, g, last_g)
    be0 = g % _E
    be1 = g // _E
    return dest, row_ids, be0, be1, act


def kernel(input_val, n0_W1, n0_b1, n0_W2, n0_b2, n1_preW, n1_preb,
           n1_W1, n1_b1, n1_W2, n1_b2):
    feats = input_val[:, 0:256]
    oh0 = input_val[:, 256:264]
    oh1 = input_val[:, 264:272]

    dest, row_ids, be0, be1, act = _routing(oh0, oh1)

    f01_s = _make_sc_gather(_B, 256, _P)(feats, row_ids)
    x0_s = _mlp0_call(be0, act, f01_s, n0_W1, n0_b1, n0_W2, n0_b2)
    out_s = _mlp1_call(be1, act, x0_s, f01_s, n1_preW, n1_preb,
                       n1_W1, n1_b1, n1_W2, n1_b2)
    return _make_sc_gather(_P, 512, _B)(out_s, dest)
